# speedup vs baseline: 2.0449x; 2.0449x over previous
import math

import jax
import jax.numpy as jnp
import numpy as np
from jax import lax
from jax.experimental import pallas as pl
from jax.experimental.pallas import tpu as pltpu
from jax.experimental.pallas import tpu_sc as plsc

_D_MODEL = 128
_SEQ = 200
_BATCH = 1024
_SCALE = math.sqrt(float(_D_MODEL))

_NUM_CORES = 2
_NUM_SUBCORES = 16
_NW = _NUM_CORES * _NUM_SUBCORES
_SEQS_PER_W = _BATCH // _NW
_VREGS_PER_ROW = _D_MODEL // 16


def _make_pe():
    pe = np.zeros((_SEQ, _D_MODEL), dtype=np.float32)
    position = np.arange(0, _SEQ, dtype=np.float32)[:, None]
    div_term = np.exp(
        np.arange(0, _D_MODEL, 2, dtype=np.float32)
        * -(math.log(10000.0) / _D_MODEL)
    )
    pe[:, 0::2] = np.sin(position * div_term)
    pe[:, 1::2] = np.cos(position * div_term)
    return pe


_PE = _make_pe()


def _body(lut_hbm, idx_hbm, pe_hbm, out_hbm, pe_v, idx_v, rows_v, sem):
    wid = lax.axis_index("s") * _NUM_CORES + lax.axis_index("c")
    pltpu.sync_copy(pe_hbm, pe_v)

    @pl.loop(0, _SEQS_PER_W)
    def _seq_loop(s):
        base = (wid * _SEQS_PER_W + s) * _SEQ
        pltpu.sync_copy(idx_hbm.at[pl.ds(base, _SEQ)], idx_v)
        pltpu.async_copy(lut_hbm.at[idx_v], rows_v, sem).wait()

        @pl.loop(0, _SEQ, unroll=2)
        def _row_loop(r):
            for j in range(_VREGS_PER_ROW):
                sl = pl.ds(j * 16, 16)
                rows_v[r, sl] = rows_v[r, sl] * _SCALE + pe_v[r, sl]

        pltpu.sync_copy(rows_v, out_hbm.at[pl.ds(base, _SEQ)])


@jax.jit
def _run(lut, idx, pe):
    kern = pl.kernel(
        _body,
        out_type=jax.ShapeDtypeStruct((_BATCH * _SEQ, _D_MODEL), jnp.float32),
        mesh=plsc.VectorSubcoreMesh(
            core_axis_name="c", subcore_axis_name="s",
            num_cores=_NUM_CORES, num_subcores=_NUM_SUBCORES,
        ),
        scratch_types=[
            pltpu.VMEM((_SEQ, _D_MODEL), jnp.float32),
            pltpu.VMEM((_SEQ,), jnp.int32),
            pltpu.VMEM((_SEQ, _D_MODEL), jnp.float32),
            pltpu.SemaphoreType.DMA,
        ],
    )
    return kern(lut, idx, pe)


def kernel(x, lut):
    idx = x.reshape(-1).astype(jnp.int32)
    pe = jnp.asarray(_PE)
    return _run(lut, idx, pe).reshape(_BATCH, _SEQ, _D_MODEL)


# speedup vs baseline: 2.8301x; 1.3840x over previous
import math

import jax
import jax.numpy as jnp
import numpy as np
from jax import lax
from jax.experimental import pallas as pl
from jax.experimental.pallas import tpu as pltpu
from jax.experimental.pallas import tpu_sc as plsc

_D_MODEL = 128
_SEQ = 200
_BATCH = 1024
_SCALE = math.sqrt(float(_D_MODEL))

_NUM_CORES = 2
_NUM_SUBCORES = 16
_NW = _NUM_CORES * _NUM_SUBCORES
_SEQS_PER_W = _BATCH // _NW
_VREGS_PER_ROW = _D_MODEL // 16


def _make_pe():
    pe = np.zeros((_SEQ, _D_MODEL), dtype=np.float32)
    position = np.arange(0, _SEQ, dtype=np.float32)[:, None]
    div_term = np.exp(
        np.arange(0, _D_MODEL, 2, dtype=np.float32)
        * -(math.log(10000.0) / _D_MODEL)
    )
    pe[:, 0::2] = np.sin(position * div_term)
    pe[:, 1::2] = np.cos(position * div_term)
    return pe


_PE = _make_pe()


_NBUF = 4
_NGROUPS = _SEQS_PER_W // _NBUF


def _body(lut_hbm, idx_hbm, pe_hbm, out_hbm, pe_v,
          idx0, idx1, idx2, idx3, rows0, rows1, rows2, rows3,
          isem0, isem1, isem2, isem3, gsem0, gsem1, gsem2, gsem3,
          ssem0, ssem1, ssem2, ssem3):
    idxb = (idx0, idx1, idx2, idx3)
    rows = (rows0, rows1, rows2, rows3)
    isem = (isem0, isem1, isem2, isem3)
    gsem = (gsem0, gsem1, gsem2, gsem3)
    ssem = (ssem0, ssem1, ssem2, ssem3)
    wid = lax.axis_index("s") * _NUM_CORES + lax.axis_index("c")
    wbase = wid * _SEQS_PER_W
    pltpu.sync_copy(pe_hbm, pe_v)

    def fire_idx(s, p):
        pltpu.async_copy(
            idx_hbm.at[pl.ds((wbase + s) * _SEQ, _SEQ)], idxb[p], isem[p])

    def wait_idx(p):
        pltpu.make_async_copy(
            idx_hbm.at[pl.ds(0, _SEQ)], idxb[p], isem[p]).wait()

    def fire_gather(p):
        pltpu.async_copy(lut_hbm.at[idxb[p]], rows[p], gsem[p])

    def wait_gather(p):
        pltpu.make_async_copy(lut_hbm.at[idxb[p]], rows[p], gsem[p]).wait()

    def fire_store(s, p):
        pltpu.async_copy(
            rows[p], out_hbm.at[pl.ds((wbase + s) * _SEQ, _SEQ)], ssem[p])

    def wait_store(p):
        pltpu.make_async_copy(
            rows[p], out_hbm.at[pl.ds(0, _SEQ)], ssem[p]).wait()

    def compute(p):
        @pl.loop(0, _SEQ, unroll=4)
        def _row_loop(r):
            for j in range(_VREGS_PER_ROW):
                sl = pl.ds(j * 16, 16)
                rows[p][r, sl] = rows[p][r, sl] * _SCALE + pe_v[r, sl]

    fire_idx(0, 0)
    fire_idx(1, 1)
    fire_idx(2, 2)
    wait_idx(0)
    fire_gather(0)
    wait_idx(1)
    fire_gather(1)

    @pl.loop(0, _NGROUPS)
    def _group(g):
        for b in range(_NBUF):
            s = 4 * g + b
            q = (b + 2) % _NBUF
            wait_gather(b)
            compute(b)
            fire_store(s, b)
            if b == 0:
                fire_idx(s + 3, 3)
            else:
                @pl.when(g < _NGROUPS - 1)
                def _():
                    fire_idx(s + 3, (b + 3) % _NBUF)
            if b < 2:
                @pl.when(g >= 1)
                def _():
                    wait_store(q)
                wait_idx(q)
                fire_gather(q)
            else:
                wait_store(q)

                @pl.when(g < _NGROUPS - 1)
                def _():
                    wait_idx(q)
                    fire_gather(q)

    wait_store(2)
    wait_store(3)


@jax.jit
def _run(lut, idx, pe):
    kern = pl.kernel(
        _body,
        out_type=jax.ShapeDtypeStruct((_BATCH * _SEQ, _D_MODEL), jnp.float32),
        mesh=plsc.VectorSubcoreMesh(
            core_axis_name="c", subcore_axis_name="s",
            num_cores=_NUM_CORES, num_subcores=_NUM_SUBCORES,
        ),
        scratch_types=(
            [pltpu.VMEM((_SEQ, _D_MODEL), jnp.float32)]
            + [pltpu.VMEM((_SEQ,), jnp.int32)] * _NBUF
            + [pltpu.VMEM((_SEQ, _D_MODEL), jnp.float32)] * _NBUF
            + [pltpu.SemaphoreType.DMA] * (3 * _NBUF)
        ),
    )
    return kern(lut, idx, pe)


def kernel(x, lut):
    idx = x.reshape(-1).astype(jnp.int32)
    pe = jnp.asarray(_PE)
    return _run(lut, idx, pe).reshape(_BATCH, _SEQ, _D_MODEL)


# speedup vs baseline: 7.2369x; 2.5572x over previous
import math

import jax
import jax.numpy as jnp
import numpy as np
from jax import lax
from jax.experimental import pallas as pl
from jax.experimental.pallas import tpu as pltpu
from jax.experimental.pallas import tpu_sc as plsc

_D_MODEL = 128
_SEQ = 200
_BATCH = 1024
_SCALE = math.sqrt(float(_D_MODEL))

_NUM_CORES = 2
_NUM_SUBCORES = 16
_NW = _NUM_CORES * _NUM_SUBCORES
_SEQS_PER_W = _BATCH // _NW
_VREGS_PER_ROW = _D_MODEL // 16


def _make_pe():
    pe = np.zeros((_SEQ, _D_MODEL), dtype=np.float32)
    position = np.arange(0, _SEQ, dtype=np.float32)[:, None]
    div_term = np.exp(
        np.arange(0, _D_MODEL, 2, dtype=np.float32)
        * -(math.log(10000.0) / _D_MODEL)
    )
    pe[:, 0::2] = np.sin(position * div_term)
    pe[:, 1::2] = np.cos(position * div_term)
    return pe


_PE = _make_pe()


_NBUF = 4
_NGROUPS = _SEQS_PER_W // _NBUF


def _body(lut_hbm, idx_hbm, pe_hbm, out_hbm, pe_v,
          idx0, idx1, idx2, idx3, rows0, rows1, rows2, rows3,
          isem0, isem1, isem2, isem3, gsem0, gsem1, gsem2, gsem3,
          ssem0, ssem1, ssem2, ssem3):
    idxb = (idx0, idx1, idx2, idx3)
    rows = (rows0, rows1, rows2, rows3)
    isem = (isem0, isem1, isem2, isem3)
    gsem = (gsem0, gsem1, gsem2, gsem3)
    ssem = (ssem0, ssem1, ssem2, ssem3)
    wid = lax.axis_index("s") * _NUM_CORES + lax.axis_index("c")
    wbase = wid * _SEQS_PER_W
    pltpu.sync_copy(pe_hbm, pe_v)

    def fire_idx(s, p):
        pltpu.async_copy(
            idx_hbm.at[pl.ds((wbase + s) * _SEQ, _SEQ)], idxb[p], isem[p])

    def wait_idx(p):
        pltpu.make_async_copy(
            idx_hbm.at[pl.ds(0, _SEQ)], idxb[p], isem[p]).wait()

    def fire_gather(p):
        pltpu.async_copy(lut_hbm.at[idxb[p]], rows[p], gsem[p])

    def wait_gather(p):
        pltpu.make_async_copy(lut_hbm.at[idxb[p]], rows[p], gsem[p]).wait()

    def fire_store(s, p):
        pltpu.async_copy(
            rows[p], out_hbm.at[pl.ds((wbase + s) * _SEQ, _SEQ)], ssem[p])

    def wait_store(p):
        pltpu.make_async_copy(
            rows[p], out_hbm.at[pl.ds(0, _SEQ)], ssem[p]).wait()

    def compute(p):
        @plsc.parallel_loop(0, _SEQ, unroll=4)
        def _row_loop(r):
            for j in range(_VREGS_PER_ROW):
                sl = pl.ds(j * 16, 16)
                rows[p][r, sl] = rows[p][r, sl] * _SCALE + pe_v[r, sl]

    fire_idx(0, 0)
    fire_idx(1, 1)
    fire_idx(2, 2)
    wait_idx(0)
    fire_gather(0)
    wait_idx(1)
    fire_gather(1)

    @pl.loop(0, _NGROUPS)
    def _group(g):
        for b in range(_NBUF):
            s = 4 * g + b
            q = (b + 2) % _NBUF
            wait_gather(b)
            compute(b)
            fire_store(s, b)
            if b == 0:
                fire_idx(s + 3, 3)
            else:
                @pl.when(g < _NGROUPS - 1)
                def _():
                    fire_idx(s + 3, (b + 3) % _NBUF)
            if b < 2:
                @pl.when(g >= 1)
                def _():
                    wait_store(q)
                wait_idx(q)
                fire_gather(q)
            else:
                wait_store(q)

                @pl.when(g < _NGROUPS - 1)
                def _():
                    wait_idx(q)
                    fire_gather(q)

    wait_store(2)
    wait_store(3)


@jax.jit
def _run(lut, idx, pe):
    kern = pl.kernel(
        _body,
        out_type=jax.ShapeDtypeStruct((_BATCH * _SEQ, _D_MODEL), jnp.float32),
        mesh=plsc.VectorSubcoreMesh(
            core_axis_name="c", subcore_axis_name="s",
            num_cores=_NUM_CORES, num_subcores=_NUM_SUBCORES,
        ),
        scratch_types=(
            [pltpu.VMEM((_SEQ, _D_MODEL), jnp.float32)]
            + [pltpu.VMEM((_SEQ,), jnp.int32)] * _NBUF
            + [pltpu.VMEM((_SEQ, _D_MODEL), jnp.float32)] * _NBUF
            + [pltpu.SemaphoreType.DMA] * (3 * _NBUF)
        ),
    )
    return kern(lut, idx, pe)


def kernel(x, lut):
    idx = x.reshape(-1).astype(jnp.int32)
    pe = jnp.asarray(_PE)
    return _run(lut, idx, pe).reshape(_BATCH, _SEQ, _D_MODEL)
